# baseline (speedup 1.0000x reference)
import functools
import jax
import jax.numpy as jnp
from jax import lax
from jax.experimental import pallas as pl
from jax.experimental.pallas import tpu as pltpu
from jax.experimental.pallas import tpu_sc as plsc

N = 10000
E = 320000
D = 128
NP = 10240
NC, NS = 2, 16
NT = NC * NS
C = 80
NCHT = E // C
CHT = NCHT // NT
RPT = NP // NS


def _sc_aggregate(feat, src2, dst2, zeros, with_cnt):
    mesh = plsc.VectorSubcoreMesh(core_axis_name="c", subcore_axis_name="s",
                                  num_cores=NC, num_subcores=NS)
    nbuf, ibr = (3, 6) if with_cnt else (4, 8)
    out_type = [jax.ShapeDtypeStruct((NC, NP, D), jnp.float32)]
    scratch = [
        pltpu.VMEM((ibr, C), jnp.int32),
        pltpu.VMEM((ibr, C), jnp.int32),
    ] + [pltpu.VMEM((C, D), jnp.float32)] * nbuf \
      + [pltpu.SemaphoreType.DMA] * (2 * nbuf + ibr) + [
        pltpu.VMEM_SHARED((NP, D), jnp.float32),
    ]
    if with_cnt:
        out_type.append(jax.ShapeDtypeStruct((NT, NP), jnp.float32))
        scratch.append(pltpu.VMEM((NP,), jnp.float32))

    @functools.partial(
        pl.kernel, mesh=mesh, out_type=out_type, scratch_types=scratch,
        compiler_params=pltpu.CompilerParams(needs_layout_passes=False),
    )
    def k(feat_h, src_h, dst_h, z_h, out_h, *rest):
        rest = list(rest)
        cnt_h = rest.pop(0) if with_cnt else None
        sidx = rest.pop(0)
        didx = rest.pop(0)
        rows = [rest.pop(0) for _ in range(nbuf)]
        gs = [rest.pop(0) for _ in range(nbuf)]
        ss = [rest.pop(0) for _ in range(nbuf)]
        isem = [rest.pop(0) for _ in range(ibr)]
        acc = rest.pop(0)
        cntv = rest.pop(0) if with_cnt else None
        c = lax.axis_index("c")
        s = lax.axis_index("s")
        r0 = s * RPT
        pltpu.sync_copy(z_h.at[pl.ds(r0, RPT)], acc.at[pl.ds(r0, RPT)])
        if with_cnt:
            def zc(i, carry):
                cntv[pl.ds(i * 16, 16)] = jnp.zeros((16,), jnp.float32)
                return carry
            lax.fori_loop(0, NP // 16, zc, 0)
        plsc.subcore_barrier()

        ebase = (c * NS + s) * CHT
        nch = CHT

        def si_d(ch, ib):
            pltpu.async_copy(src_h.at[ebase + ch], sidx.at[ib], isem[ib])
            pltpu.async_copy(dst_h.at[ebase + ch], didx.at[ib], isem[ib])

        def wi(ch, ib):
            pltpu.make_async_copy(src_h.at[ebase + ch], sidx.at[ib],
                                  isem[ib]).wait()
            pltpu.make_async_copy(dst_h.at[ebase + ch], didx.at[ib],
                                  isem[ib]).wait()

        def sg(ib, b):
            pltpu.async_copy(feat_h.at[sidx.at[ib]], rows[b], gs[b])

        def wg(ib, b):
            pltpu.make_async_copy(feat_h.at[sidx.at[ib]], rows[b],
                                  gs[b]).wait()

        def sc_(ib, b):
            pltpu.async_copy(rows[b], acc.at[didx.at[ib]], ss[b], add=True)

        def ws(ib, b):
            pltpu.make_async_copy(rows[b], acc.at[didx.at[ib]], ss[b]).wait()

        def cu(ib):
            if with_cnt:
                ones = jnp.ones((16,), jnp.float32)
                for g in range(C // 16):
                    idx16 = didx[ib, pl.ds(g * 16, 16)]
                    plsc.addupdate_scatter(cntv, [idx16], ones)

        def slot(ch, off):
            b = off % nbuf
            ib = off % ibr
            wg(ib, b)
            sc_(ib, b)
            if off >= nbuf - 1:
                ws((off - nbuf + 1) % ibr, (off - nbuf + 1) % nbuf)
            if off + 1 < nch:
                wi(ch + 1, (off + 1) % ibr)
                sg((off + 1) % ibr, (off + 1) % nbuf)
            if off + 3 < nch:
                si_d(ch + 3, (off + 3) % ibr)
            cu(ib)

        U = ibr
        assert U % nbuf == 0
        si_d(0, 0); si_d(1, 1); si_d(2, 2)
        wi(0, 0)
        sg(0, 0)
        for ch in range(U):
            slot(ch, ch)
        T = (nch - 4 - U) // U

        def group(g, carry):
            base = U + g * U
            for o in range(U):
                slot(base + o, U + o)
            return carry

        lax.fori_loop(0, T, group, 0)
        for ch in range(U + T * U, nch):
            slot(ch, ch)
        for ch in range(nch - nbuf + 1, nch):
            ws(ch % ibr, ch % nbuf)

        plsc.subcore_barrier()
        pltpu.sync_copy(acc.at[pl.ds(r0, RPT)],
                        out_h.at[c].at[pl.ds(r0, RPT)])
        if with_cnt:
            pltpu.sync_copy(cntv, cnt_h.at[c * NS + s])

    res = k(feat, src2, dst2, zeros)
    if with_cnt:
        return res[0], res[1]
    return res[0] if isinstance(res, (list, tuple)) else res


def _tc_dense(p, cnt, root, WlT, bl, WrT, relu):
    B = 400

    def body(p0_r, p1_r, c_r, x_r, wl_r, bl_r, wr_r, o_r):
        ssum = p0_r[0] + p1_r[0]
        cs = jnp.sum(c_r[...], axis=1, keepdims=True)
        mean = ssum / jnp.maximum(cs, 1.0)
        h = (jnp.dot(mean, wl_r[...], preferred_element_type=jnp.float32)
             + jnp.dot(x_r[...], wr_r[...],
                       preferred_element_type=jnp.float32)
             + bl_r[...])
        if relu:
            h = jnp.maximum(h, 0.0)
        o_r[...] = h

    return pl.pallas_call(
        body,
        grid=(N // B,),
        in_specs=[
            pl.BlockSpec((1, B, D), lambda i: (0, i, 0)),
            pl.BlockSpec((1, B, D), lambda i: (1, i, 0)),
            pl.BlockSpec((B, NT), lambda i: (i, 0)),
            pl.BlockSpec((B, D), lambda i: (i, 0)),
            pl.BlockSpec((D, D), lambda i: (0, 0)),
            pl.BlockSpec((1, D), lambda i: (0, 0)),
            pl.BlockSpec((D, D), lambda i: (0, 0)),
        ],
        out_specs=pl.BlockSpec((B, D), lambda i: (i, 0)),
        out_shape=jax.ShapeDtypeStruct((N, D), jnp.float32),
    )(p, p, cnt, root, WlT, bl, WrT)


def kernel(x, edge_index, W1l, b1l, W1r, W2l, b2l, W2r):
    src = edge_index[0].reshape(NCHT, C)
    dst = edge_index[1].reshape(NCHT, C)
    zeros = jnp.zeros((NP, D), jnp.float32)

    p, cnt = _sc_aggregate(x, src, dst, zeros, True)
    cntT = cnt.T
    h = _tc_dense(p, cntT, x, W1l.T, b1l[None, :], W1r.T, True)
    q = _sc_aggregate(h, src, dst, zeros, False)
    return _tc_dense(q, cntT, h, W2l.T, b2l[None, :], W2r.T, False)


# speedup vs baseline: 1.1740x; 1.1740x over previous
import functools
import math

import jax
import jax.numpy as jnp
from jax import lax
from jax.experimental import pallas as pl
from jax.experimental.pallas import tpu as pltpu
from jax.experimental.pallas import tpu_sc as plsc

N = 10000
E = 320000
D = 128
NP = 10000
NC, NS = 2, 16
NT = NC * NS
C = 128
NCHT = E // C
CHA = NCHT // NT
NX = NCHT - CHA * NT
RPT = 624
RPTL = NP - RPT * (NS - 1)


def _sc_aggregate(feat, src2, dst2, zeros, with_cnt):
    mesh = plsc.VectorSubcoreMesh(core_axis_name="c", subcore_axis_name="s",
                                  num_cores=NC, num_subcores=NS)
    nbuf, ibr = (2, 4)
    out_type = [jax.ShapeDtypeStruct((NC, NP, D), jnp.float32)]
    scratch = [
        pltpu.VMEM((2 * ibr, C), jnp.int32),
    ] + [pltpu.VMEM((C, D), jnp.float32)] * nbuf \
      + [pltpu.SemaphoreType.DMA] * (2 * nbuf + ibr) + [
        pltpu.VMEM_SHARED((NP, D), jnp.float32),
    ]
    if with_cnt:
        out_type.append(jax.ShapeDtypeStruct((NT, NP), jnp.float32))
        scratch.append(pltpu.VMEM((NP,), jnp.float32))

    @functools.partial(
        pl.kernel, mesh=mesh, out_type=out_type, scratch_types=scratch,
        compiler_params=pltpu.CompilerParams(needs_layout_passes=False),
    )
    def k(feat_h, src_h, dst_h, z_h, out_h, *rest):
        rest = list(rest)
        cnt_h = rest.pop(0) if with_cnt else None
        idxb = rest.pop(0)
        rows = [rest.pop(0) for _ in range(nbuf)]
        gs = [rest.pop(0) for _ in range(nbuf)]
        ss = [rest.pop(0) for _ in range(nbuf)]
        isem = [rest.pop(0) for _ in range(ibr)]
        acc = rest.pop(0)
        cntv = rest.pop(0) if with_cnt else None
        c = lax.axis_index("c")
        s = lax.axis_index("s")
        r0 = s * RPT

        @pl.when(s < NS - 1)
        def _():
            pltpu.sync_copy(z_h.at[pl.ds(r0, RPT)], acc.at[pl.ds(r0, RPT)])

        @pl.when(s == NS - 1)
        def _():
            pltpu.sync_copy(z_h.at[pl.ds((NS - 1) * RPT, RPTL)],
                            acc.at[pl.ds((NS - 1) * RPT, RPTL)])
        if with_cnt:
            def zc(i, carry):
                cntv[pl.ds(i * 16, 16)] = jnp.zeros((16,), jnp.float32)
                return carry
            lax.fori_loop(0, NP // 16, zc, 0)
        plsc.subcore_barrier()

        wid = c * NS + s

        def run(ebase, nch):
            def si_d(ch, ib):
                pltpu.async_copy(src_h.at[ebase + ch], idxb.at[2 * ib],
                                 isem[ib])
                pltpu.async_copy(dst_h.at[ebase + ch], idxb.at[2 * ib + 1],
                                 isem[ib])

            def wi(ch, ib):
                pltpu.make_async_copy(src_h.at[ebase + ch], idxb.at[2 * ib],
                                      isem[ib]).wait()
                pltpu.make_async_copy(dst_h.at[ebase + ch],
                                      idxb.at[2 * ib + 1], isem[ib]).wait()

            def sg(ib, b):
                pltpu.async_copy(feat_h.at[idxb.at[2 * ib]], rows[b], gs[b])

            def wg(ib, b):
                pltpu.make_async_copy(feat_h.at[idxb.at[2 * ib]], rows[b],
                                      gs[b]).wait()

            def sc_(ib, b):
                pltpu.async_copy(rows[b], acc.at[idxb.at[2 * ib + 1]],
                                 ss[b], add=True)

            def ws(ib, b):
                pltpu.make_async_copy(rows[b], acc.at[idxb.at[2 * ib + 1]],
                                      ss[b]).wait()

            def cu(ib):
                if with_cnt:
                    ones = jnp.ones((16,), jnp.float32)
                    for g in range(C // 16):
                        idx16 = idxb[2 * ib + 1, pl.ds(g * 16, 16)]
                        plsc.addupdate_scatter(cntv, [idx16], ones)

            LA = ibr - nbuf

            def slot(ch, off):
                b = off % nbuf
                ib = off % ibr
                wg(ib, b)
                sc_(ib, b)
                if off >= nbuf - 1:
                    ws((off - nbuf + 1) % ibr, (off - nbuf + 1) % nbuf)
                if off + 1 < nch:
                    wi(ch + 1, (off + 1) % ibr)
                    sg((off + 1) % ibr, (off + 1) % nbuf)
                if off + LA < nch:
                    si_d(ch + LA, (off + LA) % ibr)
                cu(ib)

            U = nbuf * ibr // math.gcd(nbuf, ibr)
            for i in range(LA):
                si_d(i, i)
            wi(0, 0)
            sg(0, 0)
            for ch in range(U):
                slot(ch, ch)
            T = (nch - 4 - U) // U

            def group(g, carry):
                base = U + g * U
                for o in range(U):
                    slot(base + o, U + o)
                return carry

            lax.fori_loop(0, T, group, 0)
            for ch in range(U + T * U, nch):
                slot(ch, ch)
            for ch in range(nch - nbuf + 1, nch):
                ws(ch % ibr, ch % nbuf)

        @pl.when(wid < NT - NX)
        def _():
            run(wid * CHA, CHA)

        @pl.when(wid >= NT - NX)
        def _():
            run((NT - NX) * CHA + (wid - (NT - NX)) * (CHA + 1), CHA + 1)

        plsc.subcore_barrier()

        @pl.when(s < NS - 1)
        def _():
            pltpu.sync_copy(acc.at[pl.ds(r0, RPT)],
                            out_h.at[c].at[pl.ds(r0, RPT)])

        @pl.when(s == NS - 1)
        def _():
            pltpu.sync_copy(acc.at[pl.ds((NS - 1) * RPT, RPTL)],
                            out_h.at[c].at[pl.ds((NS - 1) * RPT, RPTL)])

        if with_cnt:
            pltpu.sync_copy(cntv, cnt_h.at[wid])

    res = k(feat, src2, dst2, zeros)
    if with_cnt:
        return res[0], res[1]
    return res[0] if isinstance(res, (list, tuple)) else res


def _tc_dense(p, cnt, root, WlT, bl, WrT, relu):
    B = 400

    def body(p0_r, p1_r, c_r, x_r, wl_r, bl_r, wr_r, o_r):
        ssum = p0_r[0] + p1_r[0]
        cs = jnp.sum(c_r[...], axis=1, keepdims=True)
        mean = ssum / jnp.maximum(cs, 1.0)
        h = (jnp.dot(mean, wl_r[...], preferred_element_type=jnp.float32)
             + jnp.dot(x_r[...], wr_r[...],
                       preferred_element_type=jnp.float32)
             + bl_r[...])
        if relu:
            h = jnp.maximum(h, 0.0)
        o_r[...] = h

    return pl.pallas_call(
        body,
        grid=(N // B,),
        in_specs=[
            pl.BlockSpec((1, B, D), lambda i: (0, i, 0)),
            pl.BlockSpec((1, B, D), lambda i: (1, i, 0)),
            pl.BlockSpec((B, NT), lambda i: (i, 0)),
            pl.BlockSpec((B, D), lambda i: (i, 0)),
            pl.BlockSpec((D, D), lambda i: (0, 0)),
            pl.BlockSpec((1, D), lambda i: (0, 0)),
            pl.BlockSpec((D, D), lambda i: (0, 0)),
        ],
        out_specs=pl.BlockSpec((B, D), lambda i: (i, 0)),
        out_shape=jax.ShapeDtypeStruct((N, D), jnp.float32),
    )(p, p, cnt, root, WlT, bl, WrT)


def kernel(x, edge_index, W1l, b1l, W1r, W2l, b2l, W2r):
    src = edge_index[0].reshape(NCHT, C)
    dst = edge_index[1].reshape(NCHT, C)
    zeros = jnp.zeros((NP, D), jnp.float32)

    p, cnt = _sc_aggregate(x, src, dst, zeros, True)
    cntT = cnt.T
    h = _tc_dense(p, cntT, x, W1l.T, b1l[None, :], W1r.T, True)
    q = _sc_aggregate(h, src, dst, zeros, False)
    return _tc_dense(q, cntT, h, W2l.T, b2l[None, :], W2r.T, False)
